# baseline (speedup 1.0000x reference)
import functools

import jax
import jax.numpy as jnp
from jax import lax
from jax.experimental import pallas as pl
from jax.experimental.pallas import tpu as pltpu
from jax.experimental.pallas import tpu_sc as plsc

MAX_IDX = 8191
NROW = 4096
NCOL = 200
D = 64
V = 8192

NC = 2
NS = 16
NW = NC * NS
DSLAB = 8
NOCT = D // DSLAB
TPG = NW // NOCT
CROWS = 8
NTR = NCOL // CROWS
CPT = CROWS // TPG
QW = 2048
NQ = NROW // QW
LANES = 16
RB_PER_Q = QW // LANES
UNROLL = 16


def _make_kernel():
  mesh = plsc.VectorSubcoreMesh(core_axis_name="c", subcore_axis_name="s")

  @functools.partial(
      pl.kernel,
      mesh=mesh,
      out_type=jax.ShapeDtypeStruct((NCOL, D, NROW), jnp.float32),
      compiler_params=pltpu.CompilerParams(needs_layout_passes=False),
      scratch_types=[
          pltpu.VMEM((DSLAB, V), jnp.float32),
          pltpu.VMEM((CPT, NROW), jnp.int32),
          pltpu.VMEM((CPT, NROW), jnp.int32),
          pltpu.VMEM((DSLAB, QW), jnp.float32),
          pltpu.VMEM((DSLAB, QW), jnp.float32),
          pltpu.SemaphoreType.DMA,
          pltpu.SemaphoreType.DMA,
          pltpu.SemaphoreType.DMA,
          pltpu.SemaphoreType.DMA,
      ],
  )
  def emb_kernel(idx_hbm, table_hbm, out_hbm, slab, ig0, ig1, ob0, ob1,
                 i_sem0, i_sem1, s_sem0, s_sem1):
    cid = lax.axis_index("c")
    sid = lax.axis_index("s")
    wid = sid * NC + cid
    oct_ = wid // TPG
    sub = wid % TPG
    d0 = oct_ * DSLAB
    igs = (ig0, ig1)
    i_sems = (i_sem0, i_sem1)
    obufs = (ob0, ob1)
    s_sems = (s_sem0, s_sem1)
    dvecs = [jnp.full((LANES,), d, jnp.int32) for d in range(DSLAB)]

    pltpu.sync_copy(table_hbm.at[pl.ds(d0, DSLAB)], slab)

    def idx_copy(tr, h):
      return pltpu.make_async_copy(
          idx_hbm.at[pl.ds(tr * CROWS + sub * CPT, CPT)], igs[h], i_sems[h])

    def compute_tr(tr, ig):
      for ci in range(CPT):
        c = tr * CROWS + sub * CPT + ci
        for q in range(NQ):
          slot = (ci * NQ + q) % 2
          ob = obufs[slot]
          sem = s_sems[slot]

          def wait_prev():
            pltpu.make_async_copy(
                ob, out_hbm.at[c, pl.ds(d0, DSLAB), pl.ds(q * QW, QW)], sem
            ).wait()

          if ci * NQ + q >= 2:
            wait_prev()
          else:
            @pl.when(tr > 0)
            def _():
              wait_prev()

          def rb_body(rb):
            roff = rb * LANES
            idxv = jnp.minimum(
                ig[ci, pl.ds(q * QW + roff, LANES)], MAX_IDX)
            for d in range(DSLAB):
              ob[d, pl.ds(roff, LANES)] = plsc.load_gather(
                  slab, [dvecs[d], idxv])

          plsc.parallel_loop(0, RB_PER_Q, 1, unroll=UNROLL)(rb_body)
          pltpu.async_copy(
              ob, out_hbm.at[c, pl.ds(d0, DSLAB), pl.ds(q * QW, QW)], sem)

    idx_copy(0, 0).start()
    idx_copy(1, 1).start()

    def group_body(g, carry):
      for h in range(2):
        tr = 2 * g + h
        idx_copy(tr, h).wait()
        compute_tr(tr, igs[h])

        @pl.when(tr + 2 < NTR)
        def _():
          idx_copy(tr + 2, h).start()
      return carry

    lax.fori_loop(0, NTR // 2, group_body, 0)

    idx_copy(NTR - 1, 0).wait()
    compute_tr(NTR - 1, igs[0])

    for slot in range(2):
      pltpu.make_async_copy(
          obufs[slot], out_hbm.at[0, pl.ds(d0, DSLAB), pl.ds(0, QW)],
          s_sems[slot],
      ).wait()

  return emb_kernel


_EMB_KERNEL = _make_kernel()


@jax.jit
def kernel(input, table):
  out_t = _EMB_KERNEL(input.T, table.T)
  return out_t.transpose(2, 0, 1)


# speedup vs baseline: 1.0889x; 1.0889x over previous
import functools

import jax
import jax.numpy as jnp
from jax import lax
from jax.experimental import pallas as pl
from jax.experimental.pallas import tpu as pltpu
from jax.experimental.pallas import tpu_sc as plsc

MAX_IDX = 8191
NROW = 4096
NCOL = 200
D = 64
V = 8192

NC = 2
NS = 16
NW = NC * NS
DSLAB = 8
NOCT = D // DSLAB
TPG = NW // NOCT
CROWS = 8
NTR = NCOL // CROWS
CPT = CROWS // TPG
QW = 1024
NQ = NROW // QW
LANES = 16
RB_PER_Q = QW // LANES
UNROLL = 8


def _make_kernel():
  mesh = plsc.VectorSubcoreMesh(core_axis_name="c", subcore_axis_name="s")

  @functools.partial(
      pl.kernel,
      mesh=mesh,
      out_type=jax.ShapeDtypeStruct((NCOL, D, NROW), jnp.float32),
      compiler_params=pltpu.CompilerParams(needs_layout_passes=False),
      scratch_types=[
          pltpu.VMEM((DSLAB, V), jnp.float32),
          pltpu.VMEM((CPT, NROW), jnp.int32),
          pltpu.VMEM((CPT, NROW), jnp.int32),
          pltpu.VMEM((DSLAB, QW), jnp.float32),
          pltpu.VMEM((DSLAB, QW), jnp.float32),
          pltpu.SemaphoreType.DMA,
          pltpu.SemaphoreType.DMA,
          pltpu.SemaphoreType.DMA,
          pltpu.SemaphoreType.DMA,
      ],
  )
  def emb_kernel(idx_hbm, table_hbm, out_hbm, slab, ig0, ig1, ob0, ob1,
                 i_sem0, i_sem1, s_sem0, s_sem1):
    cid = lax.axis_index("c")
    sid = lax.axis_index("s")
    wid = sid * NC + cid
    oct_ = wid // TPG
    sub = wid % TPG
    d0 = oct_ * DSLAB
    igs = (ig0, ig1)
    i_sems = (i_sem0, i_sem1)
    obufs = (ob0, ob1)
    s_sems = (s_sem0, s_sem1)
    dvecs = [jnp.full((LANES,), d, jnp.int32) for d in range(DSLAB)]

    pltpu.sync_copy(table_hbm.at[pl.ds(d0, DSLAB)], slab)

    def idx_copy(tr, h):
      return pltpu.make_async_copy(
          idx_hbm.at[pl.ds(tr * CROWS + sub * CPT, CPT)], igs[h], i_sems[h])

    def compute_tr(tr, ig):
      for ci in range(CPT):
        c = tr * CROWS + sub * CPT + ci
        for q in range(NQ):
          slot = (ci * NQ + q) % 2
          ob = obufs[slot]
          sem = s_sems[slot]

          def wait_prev():
            pltpu.make_async_copy(
                ob, out_hbm.at[c, pl.ds(d0, DSLAB), pl.ds(q * QW, QW)], sem
            ).wait()

          if ci * NQ + q >= 2:
            wait_prev()
          else:
            @pl.when(tr > 0)
            def _():
              wait_prev()

          def rb_body(rb):
            roff = rb * LANES
            idxv = jnp.minimum(
                ig[ci, pl.ds(q * QW + roff, LANES)], MAX_IDX)
            for d in range(DSLAB):
              ob[d, pl.ds(roff, LANES)] = plsc.load_gather(
                  slab, [dvecs[d], idxv])

          plsc.parallel_loop(0, RB_PER_Q, 1, unroll=UNROLL)(rb_body)
          pltpu.async_copy(
              ob, out_hbm.at[c, pl.ds(d0, DSLAB), pl.ds(q * QW, QW)], sem)

    idx_copy(0, 0).start()
    idx_copy(1, 1).start()

    def group_body(g, carry):
      for h in range(2):
        tr = 2 * g + h
        idx_copy(tr, h).wait()
        compute_tr(tr, igs[h])

        @pl.when(tr + 2 < NTR)
        def _():
          idx_copy(tr + 2, h).start()
      return carry

    lax.fori_loop(0, NTR // 2, group_body, 0)

    idx_copy(NTR - 1, 0).wait()
    compute_tr(NTR - 1, igs[0])

    for slot in range(2):
      pltpu.make_async_copy(
          obufs[slot], out_hbm.at[0, pl.ds(d0, DSLAB), pl.ds(0, QW)],
          s_sems[slot],
      ).wait()

  return emb_kernel


_EMB_KERNEL = _make_kernel()


@jax.jit
def kernel(input, table):
  out_t = _EMB_KERNEL(input.T, table.T)
  return out_t.transpose(2, 0, 1)
